# Initial kernel scaffold; baseline (speedup 1.0000x reference)
import jax
import jax.numpy as jnp
from jax.experimental import pallas as pl


def kernel(sp_rows, sp_cols, sp_values, dense, dense_vector):
    raise NotImplementedError("write your pallas kernel here")


# speedup vs baseline: 10.5105x; 10.5105x over previous
import jax
import jax.numpy as jnp
from jax import lax
from jax.experimental import pallas as pl
from jax.experimental.pallas import tpu as pltpu
from jax.experimental.pallas import tpu_sc as plsc

B = 16384
F = 256
BF = B * F

NC = 2
NS = 16

CHUNKS_PER_SC = 2
N_CHUNKS = NC * CHUNKS_PER_SC
CHUNK_WORDS = BF // N_CHUNKS
SLICE = CHUNK_WORDS // NS
STAGE = 4096

IDX_ROW = 128
BLK = 16


def _sc_body(rows_h, cols_h, vals_h, dense_h, dv_h, out_h,
             idx_v, val_v, midx_v, mval_v, stage_v, dv_v, chunk_sh):
    nblk = idx_v.shape[0]
    cid = lax.axis_index("c")
    sid = lax.axis_index("s")

    pltpu.sync_copy(rows_h.at[sid], idx_v)
    pltpu.sync_copy(vals_h.at[sid], val_v)
    pltpu.sync_copy(dv_h, dv_v)

    def _idx_outer(t, _):
        pltpu.sync_copy(cols_h.at[sid, pl.ds(t * BLK, BLK)], midx_v)
        for jj in range(BLK):
            j = t * BLK + jj
            for k in range(IDX_ROW // 16):
                sl = pl.ds(k * 16, 16)
                idx_v[j, sl] = idx_v[j, sl] * F + midx_v[jj, sl]
        return 0
    lax.fori_loop(0, nblk // BLK, _idx_outer, 0)

    for p in range(CHUNKS_PER_SC):
        chunk = cid * CHUNKS_PER_SC + p
        lo = chunk * CHUNK_WORDS

        pltpu.sync_copy(dense_h.at[pl.ds(lo + sid * SLICE, SLICE)],
                        chunk_sh.at[pl.ds(sid * SLICE, SLICE)])
        plsc.subcore_barrier()

        def _blk_body(t, _):
            for jj in range(BLK):
                j = t * BLK + jj
                for k in range(IDX_ROW // 16):
                    sl = pl.ds(k * 16, 16)
                    local = idx_v[j, sl] - lo
                    ok = (local >= 0) & (local < CHUNK_WORDS)
                    dummy = lax.iota(jnp.int32, 16) + (jj * IDX_ROW + k * 16)
                    midx_v[jj, sl] = jnp.where(ok, local, dummy)
                    mval_v[jj, sl] = jnp.where(ok, val_v[j, sl], 0.0)
            for jj in range(BLK):
                pltpu.sync_copy(mval_v.at[jj], chunk_sh.at[midx_v.at[jj]],
                                add=True)
            return 0
        lax.fori_loop(0, nblk // BLK, _blk_body, 0)

        plsc.subcore_barrier()

        def _out_body(b, _):
            pltpu.sync_copy(chunk_sh.at[pl.ds(sid * SLICE + b * STAGE, STAGE)],
                            stage_v)
            def _add_body(r, _):
                for k in range(F // 16):
                    sl = pl.ds(r * F + k * 16, 16)
                    stage_v[sl] = stage_v[sl] + dv_v[pl.ds(k * 16, 16)]
                return 0
            lax.fori_loop(0, STAGE // F, _add_body, 0)
            pltpu.sync_copy(stage_v,
                            out_h.at[pl.ds(lo + sid * SLICE + b * STAGE, STAGE)])
            return 0
        lax.fori_loop(0, SLICE // STAGE, _out_body, 0)

        plsc.subcore_barrier()


def kernel(sp_rows, sp_cols, sp_values, dense, dense_vector):
    nnz = sp_rows.shape[0]
    grain = BLK * IDX_ROW
    e_per_tile = -(-nnz // (NS * grain)) * grain
    nblk = e_per_tile // IDX_ROW
    nnz_pad = e_per_tile * NS
    padn = nnz_pad - nnz

    pad_rows = (jnp.arange(padn, dtype=jnp.int32) * 7) % B
    rows = jnp.concatenate([sp_rows.astype(jnp.int32), pad_rows])
    cols = jnp.concatenate([sp_cols.astype(jnp.int32),
                            jnp.zeros((padn,), jnp.int32)])
    vals = jnp.concatenate([sp_values, jnp.zeros((padn,), jnp.float32)])
    rows3 = rows.reshape(NS, nblk, IDX_ROW)
    cols3 = cols.reshape(NS, nblk, IDX_ROW)
    vals3 = vals.reshape(NS, nblk, IDX_ROW)
    dense_flat = dense.reshape(BF)
    dv_flat = dense_vector.reshape(F)

    mesh = plsc.VectorSubcoreMesh(core_axis_name="c", subcore_axis_name="s")
    run = pl.kernel(
        _sc_body,
        out_type=jax.ShapeDtypeStruct((BF,), jnp.float32),
        mesh=mesh,
        scratch_types=[
            pltpu.VMEM((nblk, IDX_ROW), jnp.int32),
            pltpu.VMEM((nblk, IDX_ROW), jnp.float32),
            pltpu.VMEM((BLK, IDX_ROW), jnp.int32),
            pltpu.VMEM((BLK, IDX_ROW), jnp.float32),
            pltpu.VMEM((STAGE,), jnp.float32),
            pltpu.VMEM((F,), jnp.float32),
            pltpu.VMEM_SHARED((CHUNK_WORDS,), jnp.float32),
        ],
    )
    out = run(rows3, cols3, vals3, dense_flat, dv_flat)
    return out.reshape(B, F)


# speedup vs baseline: 12.4214x; 1.1818x over previous
import jax
import jax.numpy as jnp
from jax import lax
from jax.experimental import pallas as pl
from jax.experimental.pallas import tpu as pltpu
from jax.experimental.pallas import tpu_sc as plsc

B = 16384
F = 256
BF = B * F

NC = 2
NS = 16

CHUNKS_PER_SC = 2
N_CHUNKS = NC * CHUNKS_PER_SC
CHUNK_WORDS = BF // N_CHUNKS
SLICE = CHUNK_WORDS // NS
STAGE = 2048

IDX_ROW = 128
BLK = 8


def _sc_body(rows_h, cols_h, vals_h, dense_h, dv_h, out_h,
             idx_v, val_v, midx0, mval0, midx1, mval1, stage_v, dv_v,
             chunk_sh, sem_sc, sem_ld, sem_st):
    nblk = idx_v.shape[0]
    nb = nblk // BLK
    cid = lax.axis_index("c")
    sid = lax.axis_index("s")

    pltpu.sync_copy(rows_h.at[sid], idx_v)
    pltpu.sync_copy(vals_h.at[sid], val_v)
    pltpu.sync_copy(dv_h, dv_v)

    def _idx_outer(t, _):
        pltpu.sync_copy(cols_h.at[sid, pl.ds(t * BLK * IDX_ROW, BLK * IDX_ROW)],
                        midx0)
        for jj in range(BLK):
            j = t * BLK + jj
            for k in range(IDX_ROW // 16):
                sl = pl.ds(k * 16, 16)
                c = midx0[pl.ds(jj * IDX_ROW + k * 16, 16)]
                idx_v[j, sl] = idx_v[j, sl] * F + c
        return 0
    lax.fori_loop(0, nb, _idx_outer, 0)

    def _mask_block(t, midx, mval, lo):
        tb = lax.min(t, nb - 1)
        for jj in range(BLK):
            j = tb * BLK + jj
            for k in range(IDX_ROW // 16):
                sl = pl.ds(k * 16, 16)
                local = idx_v[j, sl] - lo
                ok = local.astype(jnp.uint32) < jnp.uint32(CHUNK_WORDS)
                dummy = lax.iota(jnp.int32, 16) + (jj * IDX_ROW + k * 16)
                osl = pl.ds(jj * IDX_ROW + k * 16, 16)
                midx[osl] = jnp.where(ok, local, dummy)
                mval[osl] = jnp.where(ok, val_v[j, sl], 0.0)

    for p in range(CHUNKS_PER_SC):
        chunk = cid * CHUNKS_PER_SC + p
        lo = chunk * CHUNK_WORDS

        _mask_block(0, midx0, mval0, lo)
        pltpu.sync_copy(dense_h.at[pl.ds(lo + sid * SLICE, SLICE)],
                        chunk_sh.at[pl.ds(sid * SLICE, SLICE)])
        plsc.subcore_barrier()

        def _blk_body(t, _):
            d0 = pltpu.async_copy(mval0, chunk_sh.at[midx0], sem_sc, add=True)
            _mask_block(2 * t + 1, midx1, mval1, lo)
            d0.wait()
            d1 = pltpu.async_copy(mval1, chunk_sh.at[midx1], sem_sc, add=True)
            _mask_block(2 * t + 2, midx0, mval0, lo)
            d1.wait()
            return 0
        lax.fori_loop(0, nb // 2, _blk_body, 0)

        plsc.subcore_barrier()

        pltpu.sync_copy(chunk_sh.at[pl.ds(sid * SLICE, STAGE)], stage_v.at[0])

        def _out_body(b, _):
            s = b & 1
            nxt = lax.min(b + 1, SLICE // STAGE - 1)
            dl = pltpu.async_copy(
                chunk_sh.at[pl.ds(sid * SLICE + nxt * STAGE, STAGE)],
                stage_v.at[s ^ 1], sem_ld)
            def _add_body(r, _):
                for k in range(F // 16):
                    sl = pl.ds(r * F + k * 16, 16)
                    stage_v[s, sl] = stage_v[s, sl] + dv_v[pl.ds(k * 16, 16)]
                return 0
            lax.fori_loop(0, STAGE // F, _add_body, 0)
            dst = pltpu.async_copy(
                stage_v.at[s],
                out_h.at[pl.ds(lo + sid * SLICE + b * STAGE, STAGE)], sem_st)
            dst.wait()
            dl.wait()
            return 0
        lax.fori_loop(0, SLICE // STAGE, _out_body, 0)

        plsc.subcore_barrier()


def kernel(sp_rows, sp_cols, sp_values, dense, dense_vector):
    nnz = sp_rows.shape[0]
    grain = 2 * BLK * IDX_ROW
    e_per_tile = -(-nnz // (NS * grain)) * grain
    nblk = e_per_tile // IDX_ROW
    nnz_pad = e_per_tile * NS
    padn = nnz_pad - nnz

    pad_rows = (jnp.arange(padn, dtype=jnp.int32) * 7) % B
    rows = jnp.concatenate([sp_rows.astype(jnp.int32), pad_rows])
    cols = jnp.concatenate([sp_cols.astype(jnp.int32),
                            jnp.zeros((padn,), jnp.int32)])
    vals = jnp.concatenate([sp_values, jnp.zeros((padn,), jnp.float32)])
    rows3 = rows.reshape(NS, nblk, IDX_ROW)
    cols3 = cols.reshape(NS, nblk * IDX_ROW)
    vals3 = vals.reshape(NS, nblk, IDX_ROW)
    dense_flat = dense.reshape(BF)
    dv_flat = dense_vector.reshape(F)

    mesh = plsc.VectorSubcoreMesh(core_axis_name="c", subcore_axis_name="s")
    run = pl.kernel(
        _sc_body,
        out_type=jax.ShapeDtypeStruct((BF,), jnp.float32),
        mesh=mesh,
        scratch_types=[
            pltpu.VMEM((nblk, IDX_ROW), jnp.int32),
            pltpu.VMEM((nblk, IDX_ROW), jnp.float32),
            pltpu.VMEM((BLK * IDX_ROW,), jnp.int32),
            pltpu.VMEM((BLK * IDX_ROW,), jnp.float32),
            pltpu.VMEM((BLK * IDX_ROW,), jnp.int32),
            pltpu.VMEM((BLK * IDX_ROW,), jnp.float32),
            pltpu.VMEM((2, STAGE), jnp.float32),
            pltpu.VMEM((F,), jnp.float32),
            pltpu.VMEM_SHARED((CHUNK_WORDS,), jnp.float32),
            pltpu.SemaphoreType.DMA,
            pltpu.SemaphoreType.DMA,
            pltpu.SemaphoreType.DMA,
        ],
    )
    out = run(rows3, cols3, vals3, dense_flat, dv_flat)
    return out.reshape(B, F)
